# baseline (speedup 1.0000x reference)
import functools

import jax
import jax.numpy as jnp
from jax import lax
from jax.experimental import pallas as pl
from jax.experimental.pallas import tpu as pltpu
from jax.experimental.pallas import tpu_sc as plsc

_N = 4096
_D = 128
_K = 10
_R = 512
_NBLK = _N // _R
_SCROWS = 512
_SCBASE = _N - _SCROWS
_NW = 32
_WROWS = _SCROWS // _NW
_CAP = 32


def _tc_body(x_ref, xt_ref, a_ref, out_ref, vthr_ref, ssc_ref, xw_out_ref,
             xw_ref, sqt_ref):
    i = pl.program_id(0)

    @pl.when(i == 0)
    def _init():
        xt = xt_ref[...]
        sqt_ref[...] = jnp.sum(xt * xt, axis=0, keepdims=True)
        xw = jnp.dot(x_ref[...], a_ref[...], preferred_element_type=jnp.float32)
        xw_ref[...] = xw
        xw_out_ref[...] = xw

    x_blk = x_ref[pl.ds(i * _R, _R), :]
    sq_blk = jnp.sum(x_blk * x_blk, axis=1, keepdims=True)
    g = jnp.dot(x_blk, xt_ref[...], preferred_element_type=jnp.float32)
    s = sq_blk + sqt_ref[...] - 2.0 * g

    row = i * _R + jax.lax.broadcasted_iota(jnp.int32, (_R, _N), 0)
    col = jax.lax.broadcasted_iota(jnp.int32, (_R, _N), 1)
    inf = jnp.float32(jnp.inf)
    s0 = jnp.where(row == col, inf, s)

    @pl.when(i == _NBLK - 1)
    def _export_scores():
        ssc_ref[...] = s0

    q = _N // 8
    p = [s0[:, j * q:(j + 1) * q] for j in range(8)]

    def cex(i_, j_):
        p[i_], p[j_] = jnp.minimum(p[i_], p[j_]), jnp.maximum(p[i_], p[j_])

    for i_, j_ in ((0, 1), (2, 3), (0, 2), (1, 3), (1, 2),
                   (4, 5), (6, 7), (4, 6), (5, 7), (5, 6),
                   (0, 4), (1, 5), (2, 6), (3, 7),
                   (2, 4), (3, 5),
                   (1, 2), (3, 4), (5, 6)):
        cex(i_, j_)

    kf = jnp.float32(_K)
    neginf = jnp.float32(-jnp.inf)
    cum = jnp.zeros((_R, 1), jnp.float32)
    v_thr = jnp.zeros((_R, 1), jnp.float32)
    cl_bk = jnp.zeros((_R, 1), jnp.float32)
    run_base = jnp.zeros((_R, 1), jnp.float32)
    m_prev = jnp.full((_R, 1), neginf)
    for _ in range(_K):
        m = jnp.min(p[0], axis=1, keepdims=True)
        eq = p[0] == m
        cnt = jnp.sum(jnp.where(eq, 1.0, 0.0), axis=1, keepdims=True)
        for j in range(7):
            p[j] = jnp.where(eq, p[j + 1], p[j])
        p[7] = jnp.where(eq, inf, p[7])
        run_base = jnp.where(m > m_prev, cum, run_base)
        newcum = cum + cnt
        sel = (cum < kf) & (newcum >= kf)
        v_thr = jnp.where(sel, m, v_thr)
        cl_bk = jnp.where(sel, run_base, cl_bk)
        cum = newcum
        m_prev = m

    vthr_ref[...] = v_thr

    lt = s0 < v_thr
    eqt = s0 == v_thr
    ce = jnp.sum(jnp.where(eqt, 1.0, 0.0), axis=1, keepdims=True)
    frac = (kf - cl_bk) / ce
    m_mask = jnp.where(lt, 1.0, jnp.where(eqt, frac, 0.0))

    out_ref[...] = jnp.dot(m_mask, xw_ref[...],
                           preferred_element_type=jnp.float32)


def _tc_call(x, xt, A):
    return pl.pallas_call(
        _tc_body,
        grid=(_NBLK,),
        in_specs=[
            pl.BlockSpec((_N, _D), lambda i: (0, 0)),
            pl.BlockSpec((_D, _N), lambda i: (0, 0)),
            pl.BlockSpec((_D, _D), lambda i: (0, 0)),
        ],
        out_specs=[
            pl.BlockSpec((_R, _D), lambda i: (i, 0)),
            pl.BlockSpec((_R, 1), lambda i: (i, 0)),
            pl.BlockSpec((_SCROWS, _N), lambda i: (0, 0)),
            pl.BlockSpec((_N, _D), lambda i: (0, 0)),
        ],
        out_shape=[
            jax.ShapeDtypeStruct((_N, _D), jnp.float32),
            jax.ShapeDtypeStruct((_N, 1), jnp.float32),
            jax.ShapeDtypeStruct((_SCROWS, _N), jnp.float32),
            jax.ShapeDtypeStruct((_N, _D), jnp.float32),
        ],
        scratch_shapes=[
            pltpu.VMEM((_N, _D), jnp.float32),
            pltpu.VMEM((1, _N), jnp.float32),
        ],
    )(x, xt, A)


def _sc_body(ssc_ref, vthr_ref, xw_ref, out_ref,
             st_ref, vthr_v, candval, candcol, glist, acc, sem):
    wid = lax.axis_index("s") * 2 + lax.axis_index("c")
    base_row = wid * _WROWS
    pltpu.sync_copy(ssc_ref.at[pl.ds(wid * _N * _WROWS, _N * _WROWS)], st_ref)
    pltpu.sync_copy(vthr_ref.at[pl.ds(base_row, _WROWS)], vthr_v)
    iot = lax.iota(jnp.int32, 16)
    inf = jnp.float32(jnp.inf)
    one = jnp.ones((16,), jnp.int32)
    zero = jnp.zeros((16,), jnp.int32)

    def _init_cv(s_, c):
        candval[pl.ds(s_ * 16, 16)] = jnp.full((16,), inf, jnp.float32)
        return c

    lax.fori_loop(0, _CAP, _init_cv, 0)

    def _init_gl(s_, c):
        glist[pl.ds(s_ * 16, 16)] = _N + ((iot + s_) & 7)
        return c

    lax.fori_loop(0, 16, _init_gl, 0)

    vthr_vec = vthr_v[...]

    def _scan(cc, cnt):
        sv = st_ref[pl.ds(cc * 16, 16)]
        msk = sv <= vthr_vec
        pos = jnp.minimum(cnt, _CAP - 1) * 16 + iot
        plsc.store_scatter(candval, [pos], sv, mask=msk)
        plsc.store_scatter(candcol, [pos], jnp.full((16,), cc, jnp.int32),
                           mask=msk)
        return cnt + jnp.where(msk, one, zero)

    lax.fori_loop(0, _N, _scan, jnp.zeros((16,), jnp.int32))

    def _nless(s_, nl):
        vals = candval[pl.ds(s_ * 16, 16)]
        return nl + jnp.where(vals < vthr_vec, one, zero)

    nless = lax.fori_loop(0, _CAP, _nless, jnp.zeros((16,), jnp.int32))
    need = _K - nless

    def _take(s_, carry):
        eqseen, gcnt = carry
        vals = candval[pl.ds(s_ * 16, 16)]
        cols = candcol[pl.ds(s_ * 16, 16)]
        ltv = vals < vthr_vec
        eqv = vals == vthr_vec
        tk = ltv | (eqv & (eqseen < need))
        gpos = jnp.minimum(gcnt * 16 + iot, 255)
        plsc.store_scatter(glist, [gpos], cols, mask=tk)
        return (eqseen + jnp.where(eqv, one, zero),
                gcnt + jnp.where(tk, one, zero))

    lax.fori_loop(0, _CAP, _take,
                  (jnp.zeros((16,), jnp.int32), jnp.zeros((16,), jnp.int32)))

    pltpu.async_copy(xw_ref.at[glist.at[pl.ds(0, 16)]], acc, sem).wait()
    descs = []
    for s_ in range(1, 16):
        descs.append(pltpu.async_copy(xw_ref.at[glist.at[pl.ds(s_ * 16, 16)]],
                                      acc, sem, add=True))
    for d_ in descs:
        d_.wait()
    pltpu.sync_copy(acc, out_ref.at[pl.ds(base_row, _WROWS), :])


def _sc_call(ssc_flat, vthr_sc, xwpad):
    mesh = plsc.VectorSubcoreMesh(core_axis_name="c", subcore_axis_name="s")
    return pl.kernel(
        _sc_body,
        out_type=jax.ShapeDtypeStruct((_SCROWS, _D), jnp.float32),
        mesh=mesh,
        scratch_types=[
            pltpu.VMEM((_N * _WROWS,), jnp.float32),
            pltpu.VMEM((_WROWS,), jnp.float32),
            pltpu.VMEM((_CAP * 16,), jnp.float32),
            pltpu.VMEM((_CAP * 16,), jnp.int32),
            pltpu.VMEM((16 * 16,), jnp.int32),
            pltpu.VMEM((16, _D), jnp.float32),
            pltpu.SemaphoreType.DMA,
        ],
        compiler_params=pltpu.CompilerParams(needs_layout_passes=False),
    )(ssc_flat, vthr_sc, xwpad)


@jax.jit
def kernel(x, A):
    xt = x.T
    tc_out, vthr, ssc, xw = _tc_call(x, xt, A)
    xwpad = jnp.concatenate([xw, jnp.zeros((8, _D), jnp.float32)], axis=0)
    sc_in = ssc.T.reshape(_N, _NW, _WROWS).swapaxes(0, 1).reshape(-1)
    sc_out = _sc_call(sc_in, vthr[_SCBASE:].reshape(_SCROWS), xwpad)
    return jnp.concatenate([tc_out[:_SCBASE], sc_out], axis=0)


# speedup vs baseline: 1.3090x; 1.3090x over previous
import functools

import jax
import jax.numpy as jnp
from jax import lax
from jax.experimental import pallas as pl
from jax.experimental.pallas import tpu as pltpu
from jax.experimental.pallas import tpu_sc as plsc

_N = 4096
_D = 128
_K = 10
_R = 512
_SCROWS = 512
_SCBASE = _N - _SCROWS
_TCROWS = _SCBASE
_NBLK = _TCROWS // _R
_NW = 32
_WROWS = _SCROWS // _NW
_CAP = 32
_UNROLL = 8


def _scores_block(x_ref, xt_ref, sqt, base):
    x_blk = x_ref[pl.ds(base, _R), :]
    sq_blk = jnp.sum(x_blk * x_blk, axis=1, keepdims=True)
    g = jnp.dot(x_blk, xt_ref[...], preferred_element_type=jnp.float32)
    s = sq_blk + sqt - 2.0 * g
    row = base + jax.lax.broadcasted_iota(jnp.int32, (_R, _N), 0)
    col = jax.lax.broadcasted_iota(jnp.int32, (_R, _N), 1)
    return jnp.where(row == col, jnp.float32(jnp.inf), s)


def _tournament(s0):
    inf = jnp.float32(jnp.inf)
    q = _N // 8
    p = [s0[:, j * q:(j + 1) * q] for j in range(8)]

    def cex(i_, j_):
        p[i_], p[j_] = jnp.minimum(p[i_], p[j_]), jnp.maximum(p[i_], p[j_])

    for i_, j_ in ((0, 1), (2, 3), (0, 2), (1, 3), (1, 2),
                   (4, 5), (6, 7), (4, 6), (5, 7), (5, 6),
                   (0, 4), (1, 5), (2, 6), (3, 7),
                   (2, 4), (3, 5),
                   (1, 2), (3, 4), (5, 6)):
        cex(i_, j_)

    kf = jnp.float32(_K)
    r = s0.shape[0]
    cum = jnp.zeros((r, 1), jnp.float32)
    v_thr = jnp.zeros((r, 1), jnp.float32)
    cl_bk = jnp.zeros((r, 1), jnp.float32)
    run_base = jnp.zeros((r, 1), jnp.float32)
    m_prev = jnp.full((r, 1), -inf)
    for _ in range(_K):
        m = jnp.min(p[0], axis=1, keepdims=True)
        eq = p[0] == m
        cnt = jnp.sum(jnp.where(eq, 1.0, 0.0), axis=1, keepdims=True)
        for j in range(7):
            p[j] = jnp.where(eq, p[j + 1], p[j])
        p[7] = jnp.where(eq, inf, p[7])
        run_base = jnp.where(m > m_prev, cum, run_base)
        newcum = cum + cnt
        sel = (cum < kf) & (newcum >= kf)
        v_thr = jnp.where(sel, m, v_thr)
        cl_bk = jnp.where(sel, run_base, cl_bk)
        cum = newcum
        m_prev = m
    return v_thr, cl_bk


def _tc1_body(x_ref, xt_ref, a_ref, ssc_ref, vthr_ref, xw_ref, sqt_ref):
    xt = xt_ref[...]
    sqt = jnp.sum(xt * xt, axis=0, keepdims=True)
    sqt_ref[...] = sqt
    xw_ref[...] = jnp.dot(x_ref[...], a_ref[...],
                          preferred_element_type=jnp.float32)
    s0 = _scores_block(x_ref, xt_ref, sqt, _SCBASE)
    ssc_ref[...] = s0
    v_thr, _ = _tournament(s0)
    vthr_ref[...] = v_thr


def _tc1_call(x, xt, A):
    return pl.pallas_call(
        _tc1_body,
        out_shape=[
            jax.ShapeDtypeStruct((_SCROWS, _N), jnp.float32),
            jax.ShapeDtypeStruct((_SCROWS, 1), jnp.float32),
            jax.ShapeDtypeStruct((_N, _D), jnp.float32),
            jax.ShapeDtypeStruct((1, _N), jnp.float32),
        ],
    )(x, xt, A)


def _tc2_body(x_ref, xt_ref, xw_ref, sqt_ref, out_ref):
    i = pl.program_id(0)
    s0 = _scores_block(x_ref, xt_ref, sqt_ref[...], i * _R)
    v_thr, cl_bk = _tournament(s0)
    kf = jnp.float32(_K)
    lt = s0 < v_thr
    eqt = s0 == v_thr
    ce = jnp.sum(jnp.where(eqt, 1.0, 0.0), axis=1, keepdims=True)
    frac = (kf - cl_bk) / ce
    m_mask = jnp.where(lt, 1.0, jnp.where(eqt, frac, 0.0))
    out_ref[...] = jnp.dot(m_mask, xw_ref[...],
                           preferred_element_type=jnp.float32)


def _tc2_call(x, xt, xw, sqt):
    return pl.pallas_call(
        _tc2_body,
        grid=(_NBLK,),
        in_specs=[
            pl.BlockSpec((_N, _D), lambda i: (0, 0)),
            pl.BlockSpec((_D, _N), lambda i: (0, 0)),
            pl.BlockSpec((_N, _D), lambda i: (0, 0)),
            pl.BlockSpec((1, _N), lambda i: (0, 0)),
        ],
        out_specs=pl.BlockSpec((_R, _D), lambda i: (i, 0)),
        out_shape=jax.ShapeDtypeStruct((_TCROWS, _D), jnp.float32),
    )(x, xt, xw, sqt)


def _sc_body(ssc_ref, vthr_ref, xw_ref, out_ref,
             st_ref, vthr_v, candval, candcol, glist, acc, sem):
    wid = lax.axis_index("s") * 2 + lax.axis_index("c")
    base_row = wid * _WROWS
    pltpu.sync_copy(ssc_ref.at[pl.ds(wid * _N * _WROWS, _N * _WROWS)], st_ref)
    pltpu.sync_copy(vthr_ref.at[pl.ds(base_row, _WROWS)], vthr_v)
    iot = lax.iota(jnp.int32, 16)
    inf = jnp.float32(jnp.inf)
    one = jnp.ones((16,), jnp.int32)
    zero = jnp.zeros((16,), jnp.int32)

    def _init_cv(s_, c):
        candval[pl.ds(s_ * 16, 16)] = jnp.full((16,), inf, jnp.float32)
        return c

    lax.fori_loop(0, _CAP, _init_cv, 0)

    def _init_gl(s_, c):
        glist[pl.ds(s_ * 16, 16)] = _N + ((iot + s_) & 7)
        return c

    lax.fori_loop(0, 16, _init_gl, 0)

    vthr_vec = vthr_v[...]

    def _scan(cc8, cnt):
        for u in range(_UNROLL):
            cc = cc8 * _UNROLL + u
            sv = st_ref[pl.ds(cc * 16, 16)]
            msk = sv <= vthr_vec
            pos = jnp.minimum(cnt, _CAP - 1) * 16 + iot
            plsc.store_scatter(candval, [pos], sv, mask=msk)
            plsc.store_scatter(candcol, [pos], jnp.full((16,), cc, jnp.int32),
                               mask=msk)
            cnt = cnt + jnp.where(msk, one, zero)
        return cnt

    lax.fori_loop(0, _N // _UNROLL, _scan, jnp.zeros((16,), jnp.int32))

    def _nless(s_, nl):
        vals = candval[pl.ds(s_ * 16, 16)]
        return nl + jnp.where(vals < vthr_vec, one, zero)

    nless = lax.fori_loop(0, _CAP, _nless, jnp.zeros((16,), jnp.int32))
    need = _K - nless

    def _take(s_, carry):
        eqseen, gcnt = carry
        vals = candval[pl.ds(s_ * 16, 16)]
        cols = candcol[pl.ds(s_ * 16, 16)]
        ltv = vals < vthr_vec
        eqv = vals == vthr_vec
        tk = ltv | (eqv & (eqseen < need))
        gpos = jnp.minimum(gcnt * 16 + iot, 255)
        plsc.store_scatter(glist, [gpos], cols, mask=tk)
        return (eqseen + jnp.where(eqv, one, zero),
                gcnt + jnp.where(tk, one, zero))

    lax.fori_loop(0, _CAP, _take,
                  (jnp.zeros((16,), jnp.int32), jnp.zeros((16,), jnp.int32)))

    pltpu.async_copy(xw_ref.at[glist.at[pl.ds(0, 16)]], acc, sem).wait()
    descs = []
    for s_ in range(1, 16):
        descs.append(pltpu.async_copy(xw_ref.at[glist.at[pl.ds(s_ * 16, 16)]],
                                      acc, sem, add=True))
    for d_ in descs:
        d_.wait()
    pltpu.sync_copy(acc, out_ref.at[pl.ds(base_row, _WROWS), :])


def _sc_call(ssc_flat, vthr_sc, xwpad):
    mesh = plsc.VectorSubcoreMesh(core_axis_name="c", subcore_axis_name="s")
    return pl.kernel(
        _sc_body,
        out_type=jax.ShapeDtypeStruct((_SCROWS, _D), jnp.float32),
        mesh=mesh,
        scratch_types=[
            pltpu.VMEM((_N * _WROWS,), jnp.float32),
            pltpu.VMEM((_WROWS,), jnp.float32),
            pltpu.VMEM((_CAP * 16,), jnp.float32),
            pltpu.VMEM((_CAP * 16,), jnp.int32),
            pltpu.VMEM((16 * 16,), jnp.int32),
            pltpu.VMEM((16, _D), jnp.float32),
            pltpu.SemaphoreType.DMA,
        ],
        compiler_params=pltpu.CompilerParams(needs_layout_passes=False),
    )(ssc_flat, vthr_sc, xwpad)


@jax.jit
def kernel(x, A):
    xt = x.T
    ssc, vthr_sc, xw, sqt = _tc1_call(x, xt, A)
    xwpad = jnp.concatenate([xw, jnp.zeros((8, _D), jnp.float32)], axis=0)
    sc_in = ssc.T.reshape(_N, _NW, _WROWS).swapaxes(0, 1).reshape(-1)
    sc_out = _sc_call(sc_in, vthr_sc.reshape(_SCROWS), xwpad)
    tc_out = _tc2_call(x, xt, xw, sqt)
    return jnp.concatenate([tc_out, sc_out], axis=0)
